# Initial kernel scaffold; baseline (speedup 1.0000x reference)
import jax
import jax.numpy as jnp
from jax.experimental import pallas as pl


def kernel(adj, embeds):
    raise NotImplementedError("write your pallas kernel here")


# speedup vs baseline: 1.2683x; 1.2683x over previous
import jax
import jax.numpy as jnp
from jax.experimental import pallas as pl
import jax.experimental.pallas.tpu as pltpu

USER_N = 8192
ITEM_N = 8192
EMB_D = 32
N_LAYER = 3
BI = 256
NI = USER_N // BI


def _lightgcn_kernel(adj_ref, embeds_ref,
                     ugcn_ref, ulat_ref, igcn_ref, ilat_ref,
                     user_cur, item_cur):
    l = pl.program_id(0)
    i = pl.program_id(1)

    @pl.when((l == 0) & (i == 0))
    def _init():
        user_cur[...] = embeds_ref[:USER_N, :]
        item_cur[...] = embeds_ref[USER_N:, :]

    a = adj_ref[...]
    u_old = user_cur[pl.ds(i * BI, BI), :]

    u_g = jax.lax.dot_general(
        a, item_cur[...], (((1,), (0,)), ((), ())),
        preferred_element_type=jnp.float32)
    ugcn_ref[0] = u_g
    u_new = u_g + u_old
    ulat_ref[0] = u_new
    user_cur[pl.ds(i * BI, BI), :] = u_new

    contrib = jax.lax.dot_general(
        a, u_old, (((0,), (0,)), ((), ())),
        preferred_element_type=jnp.float32)

    @pl.when(i == 0)
    def _first():
        igcn_ref[0] = contrib

    @pl.when(i > 0)
    def _rest():
        igcn_ref[0] += contrib

    @pl.when(i == NI - 1)
    def _finish_layer():
        i_new = igcn_ref[0] + item_cur[...]
        ilat_ref[0] = i_new
        item_cur[...] = i_new


def _run(adj, embeds):
    out_shape = [
        jax.ShapeDtypeStruct((N_LAYER, USER_N, EMB_D), jnp.float32),
        jax.ShapeDtypeStruct((N_LAYER, USER_N, EMB_D), jnp.float32),
        jax.ShapeDtypeStruct((N_LAYER, ITEM_N, EMB_D), jnp.float32),
        jax.ShapeDtypeStruct((N_LAYER, ITEM_N, EMB_D), jnp.float32),
    ]
    grid = (N_LAYER, NI)
    return pl.pallas_call(
        _lightgcn_kernel,
        grid=grid,
        in_specs=[
            pl.BlockSpec((BI, ITEM_N), lambda l, i: (i, 0)),
            pl.BlockSpec((USER_N + ITEM_N, EMB_D), lambda l, i: (0, 0)),
        ],
        out_specs=[
            pl.BlockSpec((1, BI, EMB_D), lambda l, i: (l, i, 0)),
            pl.BlockSpec((1, BI, EMB_D), lambda l, i: (l, i, 0)),
            pl.BlockSpec((1, ITEM_N, EMB_D), lambda l, i: (l, 0, 0)),
            pl.BlockSpec((1, ITEM_N, EMB_D), lambda l, i: (l, 0, 0)),
        ],
        out_shape=out_shape,
        scratch_shapes=[
            pltpu.VMEM((USER_N, EMB_D), jnp.float32),
            pltpu.VMEM((ITEM_N, EMB_D), jnp.float32),
        ],
    )(adj, embeds)


def kernel(adj, embeds):
    ugcn, ulat, igcn, ilat = _run(adj, embeds)
    lats = [embeds]
    gcn_lats = [embeds]
    for l in range(N_LAYER):
        gcn_lats.append(jnp.concatenate([ugcn[l], igcn[l]], axis=0))
        lats.append(jnp.concatenate([ulat[l], ilat[l]], axis=0))
    return (tuple(lats), tuple(gcn_lats))


# speedup vs baseline: 1.3354x; 1.0529x over previous
import jax
import jax.numpy as jnp
from jax.experimental import pallas as pl
import jax.experimental.pallas.tpu as pltpu

USER_N = 8192
ITEM_N = 8192
EMB_D = 32
N_LAYER = 3
BI = 256
NI = USER_N // BI


def _lightgcn_kernel(adj_ref, embeds_ref,
                     ugcn_ref, ulat_ref, igcn_ref, ilat_ref,
                     user_cur, item_cur, item_acc_t):
    l = pl.program_id(0)
    i = pl.program_id(1)

    @pl.when((l == 0) & (i == 0))
    def _init():
        user_cur[...] = embeds_ref[:USER_N, :]
        item_cur[...] = embeds_ref[USER_N:, :]

    a = adj_ref[...]
    u_old = user_cur[pl.ds(i * BI, BI), :]

    u_g = jax.lax.dot_general(
        a, item_cur[...], (((1,), (0,)), ((), ())),
        preferred_element_type=jnp.float32)
    ugcn_ref[0] = u_g
    u_new = u_g + u_old
    ulat_ref[0] = u_new
    user_cur[pl.ds(i * BI, BI), :] = u_new

    u_old_t = u_old.T
    contrib_t = jax.lax.dot_general(
        u_old_t, a, (((1,), (0,)), ((), ())),
        preferred_element_type=jnp.float32)

    @pl.when(i == 0)
    def _first():
        item_acc_t[...] = contrib_t

    @pl.when(i > 0)
    def _rest():
        item_acc_t[...] += contrib_t

    @pl.when(i == NI - 1)
    def _finish_layer():
        i_g = item_acc_t[...].T
        igcn_ref[0] = i_g
        i_new = i_g + item_cur[...]
        ilat_ref[0] = i_new
        item_cur[...] = i_new


def _run(adj, embeds):
    out_shape = [
        jax.ShapeDtypeStruct((N_LAYER, USER_N, EMB_D), jnp.float32),
        jax.ShapeDtypeStruct((N_LAYER, USER_N, EMB_D), jnp.float32),
        jax.ShapeDtypeStruct((N_LAYER, ITEM_N, EMB_D), jnp.float32),
        jax.ShapeDtypeStruct((N_LAYER, ITEM_N, EMB_D), jnp.float32),
    ]
    grid = (N_LAYER, NI)
    return pl.pallas_call(
        _lightgcn_kernel,
        grid=grid,
        in_specs=[
            pl.BlockSpec((BI, ITEM_N), lambda l, i: (i, 0)),
            pl.BlockSpec((USER_N + ITEM_N, EMB_D), lambda l, i: (0, 0)),
        ],
        out_specs=[
            pl.BlockSpec((1, BI, EMB_D), lambda l, i: (l, i, 0)),
            pl.BlockSpec((1, BI, EMB_D), lambda l, i: (l, i, 0)),
            pl.BlockSpec((1, ITEM_N, EMB_D), lambda l, i: (l, 0, 0)),
            pl.BlockSpec((1, ITEM_N, EMB_D), lambda l, i: (l, 0, 0)),
        ],
        out_shape=out_shape,
        scratch_shapes=[
            pltpu.VMEM((USER_N, EMB_D), jnp.float32),
            pltpu.VMEM((ITEM_N, EMB_D), jnp.float32),
            pltpu.VMEM((EMB_D, ITEM_N), jnp.float32),
        ],
    )(adj, embeds)


def kernel(adj, embeds):
    ugcn, ulat, igcn, ilat = _run(adj, embeds)
    lats = [embeds]
    gcn_lats = [embeds]
    for l in range(N_LAYER):
        gcn_lats.append(jnp.concatenate([ugcn[l], igcn[l]], axis=0))
        lats.append(jnp.concatenate([ulat[l], ilat[l]], axis=0))
    return (tuple(lats), tuple(gcn_lats))


# speedup vs baseline: 1.9278x; 1.4436x over previous
import jax
import jax.numpy as jnp
from jax.experimental import pallas as pl
import jax.experimental.pallas.tpu as pltpu

USER_N = 8192
ITEM_N = 8192
EMB_D = 32
BJ = 512
NJ = ITEM_N // BJ


def _lightgcn_kernel(adj_ref, eut_ref, eu_ref, ei_ref,
                     g1u, g2u, g3u, l1u, l2u, l3u,
                     g1i, g2i, g3i, l1i, l2i, l3i,
                     uw_acc, w1i_t, wi23_t, w3u_acc, u12_t):
    p = pl.program_id(0)
    j = pl.program_id(1)
    sl = pl.ds(j * BJ, BJ)

    @pl.when(p == 0)
    def _pass1():
        a = adj_ref[...]
        t1_t = jax.lax.dot_general(
            eut_ref[...], a, (((1,), (0,)), ((), ())),
            preferred_element_type=jnp.float32)
        w1i_t[:, sl] = t1_t
        rhs = jnp.concatenate([ei_ref[...], t1_t.T], axis=1)
        prod = jax.lax.dot_general(
            a, rhs, (((1,), (0,)), ((), ())),
            preferred_element_type=jnp.float32)

        @pl.when(j == 0)
        def _():
            uw_acc[...] = prod

        @pl.when(j > 0)
        def _():
            uw_acc[...] += prod

    @pl.when((p == 1) & (j == 0))
    def _mid():
        u12_t[...] = uw_acc[...].T

    @pl.when(p == 1)
    def _pass2():
        a = adj_ref[...]
        s_t = jax.lax.dot_general(
            u12_t[...], a, (((1,), (0,)), ((), ())),
            preferred_element_type=jnp.float32)
        wi23_t[:, sl] = s_t
        w2i_stripe = s_t[0:EMB_D, :].T
        prod2 = jax.lax.dot_general(
            a, w2i_stripe, (((1,), (0,)), ((), ())),
            preferred_element_type=jnp.float32)

        @pl.when(j == 0)
        def _():
            w3u_acc[...] = prod2

        @pl.when(j > 0)
        def _():
            w3u_acc[...] += prod2

    @pl.when(p == 2)
    def _epilogue():
        w1u = uw_acc[sl, 0:EMB_D]
        w2u = uw_acc[sl, EMB_D:2 * EMB_D]
        w3u = w3u_acc[sl, :]
        eu = eu_ref[...]
        g1u[...] = w1u
        g2u[...] = w1u + w2u
        g3u[...] = w1u + 2.0 * w2u + w3u
        l1u[...] = eu + w1u
        l2u[...] = eu + 2.0 * w1u + w2u
        l3u[...] = eu + 3.0 * w1u + 3.0 * w2u + w3u

        w1i = w1i_t[:, sl].T
        w23 = wi23_t[:, sl].T
        w2i = w23[:, 0:EMB_D]
        w3i = w23[:, EMB_D:2 * EMB_D]
        ei = ei_ref[...]
        g1i[...] = w1i
        g2i[...] = w1i + w2i
        g3i[...] = w1i + 2.0 * w2i + w3i
        l1i[...] = ei + w1i
        l2i[...] = ei + 2.0 * w1i + w2i
        l3i[...] = ei + 3.0 * w1i + 3.0 * w2i + w3i


def _run(adj, e_u_t, e_u, e_i):
    D = EMB_D
    out_sd = jax.ShapeDtypeStruct((USER_N, D), jnp.float32)
    out_shape = [out_sd] * 12

    def adj_map(p, j):
        return (0, jnp.where(p == 2, NJ - 1, j))

    def chunk_map(p, j):
        return (jnp.where(p == 2, j, 0), 0)

    return pl.pallas_call(
        _lightgcn_kernel,
        grid=(3, NJ),
        in_specs=[
            pl.BlockSpec((USER_N, BJ), adj_map),
            pl.BlockSpec((D, USER_N), lambda p, j: (0, 0)),
            pl.BlockSpec((BJ, D), chunk_map),
            pl.BlockSpec((BJ, D), lambda p, j: (j, 0)),
        ],
        out_specs=[pl.BlockSpec((BJ, D), chunk_map)] * 12,
        out_shape=out_shape,
        scratch_shapes=[
            pltpu.VMEM((USER_N, 2 * D), jnp.float32),
            pltpu.VMEM((D, ITEM_N), jnp.float32),
            pltpu.VMEM((2 * D, ITEM_N), jnp.float32),
            pltpu.VMEM((USER_N, D), jnp.float32),
            pltpu.VMEM((2 * D, USER_N), jnp.float32),
        ],
    )(adj, e_u_t, e_u, e_i)


def kernel(adj, embeds):
    e_u = embeds[:USER_N]
    e_i = embeds[USER_N:]
    e_u_t = e_u.T
    (g1u, g2u, g3u, l1u, l2u, l3u,
     g1i, g2i, g3i, l1i, l2i, l3i) = _run(adj, e_u_t, e_u, e_i)
    lats = (embeds,
            jnp.concatenate([l1u, l1i], axis=0),
            jnp.concatenate([l2u, l2i], axis=0),
            jnp.concatenate([l3u, l3i], axis=0))
    gcn_lats = (embeds,
                jnp.concatenate([g1u, g1i], axis=0),
                jnp.concatenate([g2u, g2i], axis=0),
                jnp.concatenate([g3u, g3i], axis=0))
    return (lats, gcn_lats)
